# baseline (speedup 1.0000x reference)
import numpy as np
import jax
import jax.numpy as jnp
from jax.experimental import pallas as pl
from jax.experimental.pallas import tpu as pltpu

EPSILON = 1e-6
BN_EPS = 1e-5

f32 = jnp.float32
bf16 = jnp.bfloat16

_CP = pltpu.CompilerParams(
    dimension_semantics=("parallel",),
    vmem_limit_bytes=64 * 1024 * 1024,
)

_SIN_C = (6.2830885050e+00, -4.1333250451e+01, 8.1400142117e+01,
          -7.4676222887e+01, 3.3168810291e+01)
_COS_C = (9.9999944371e-01, -1.9739034398e+01, 6.4930614506e+01,
          -8.5295987236e+01, 5.8912646156e+01, -2.1283194093e+01)


def _sincos_2pi(t):
    r = t - jnp.round(t)
    u = r * r
    s = jnp.float32(_SIN_C[-1])
    for c in _SIN_C[-2::-1]:
        s = s * u + jnp.float32(c)
    s = s * r
    c_ = jnp.float32(_COS_C[-1])
    for c in _COS_C[-2::-1]:
        c_ = c_ * u + jnp.float32(c)
    return s, c_


def _ceil_to(x, m):
    return ((x + m - 1) // m) * m


def _rows(tile, cols):
    return pl.BlockSpec((tile, cols), lambda i: (i, 0))


def _rep(shape):
    return pl.BlockSpec(shape, lambda i: (0, 0))


def _node_pre_kernel(x_ref, pos_ref, w1a_ref, w1b_ref, b1_ref, nw_ref, nb_ref,
                     a_ref, b_ref, h_ref):
    x = x_ref[...]
    pos = pos_ref[...]
    tn, d = x.shape
    a = jnp.dot(x, w1a_ref[...], preferred_element_type=f32) + b1_ref[...]
    b = jnp.dot(x, w1b_ref[...], preferred_element_type=f32)
    zr = jnp.zeros((tn, 125), f32)
    zc = jnp.zeros((tn, 4), f32)
    a_ref[...] = jnp.concatenate([a, pos, zr], axis=1)
    b_ref[...] = jnp.concatenate([b, zc, pos, zr[:, :121]], axis=1)
    h_ref[...] = jnp.dot(x.astype(bf16), nw_ref[...],
                         preferred_element_type=f32) + nb_ref[...]


def _bn_stats_kernel(nedge_ref, h1_ref, stats_ref):
    h1 = h1_ref[...]
    te, d = h1.shape
    base = pl.program_id(0) * te
    idx = base + jax.lax.broadcasted_iota(jnp.int32, (te, 1), 0)
    hm = jnp.where(idx < nedge_ref[0], h1, 0.0)
    s = jnp.sum(hm, axis=0, keepdims=True)
    ss = jnp.sum(hm * h1, axis=0, keepdims=True)
    stats_ref[...] = jnp.concatenate([s, ss, jnp.zeros((6, d), f32)], axis=0)


def _edge_kernel(hp_ref, scale_ref, shift_ref,
                 w2_ref, b2_ref, fw_ref,
                 wsin_ref, wcos_ref, wi_ref, wj_ref, pb1_ref,
                 pw2_ref, pb2_ref,
                 ea_ref, basis_ref):
    hp = hp_ref[...]
    d = scale_ref.shape[1]
    hb = jnp.maximum(hp[:, :d] * scale_ref[...] + shift_ref[...], 0.0)
    edge2d = jnp.dot(hb.astype(bf16), w2_ref[...],
                     preferred_element_type=f32) + b2_ref[...]

    rx, ry, rz = hp[:, d:d + 1], hp[:, d + 1:d + 2], hp[:, d + 2:d + 3]
    qx, qy, qz = hp[:, d + 4:d + 5], hp[:, d + 5:d + 6], hp[:, d + 6:d + 7]
    dx, dy, dz = rx - qx, ry - qy, rz - qz
    radial = dx * dx + dy * dy + dz * dz
    cx = ry * qz - rz * qy
    cy = rz * qx - rx * qz
    cz = rx * qy - ry * qx
    inv_d = pl.reciprocal(jnp.sqrt(radial) + EPSILON, approx=True)
    dx, dy, dz = dx * inv_d, dy * inv_d, dz * inv_d
    inv_c = pl.reciprocal(
        jnp.sqrt(cx * cx + cy * cy + cz * cz) + EPSILON, approx=True)
    cx, cy, cz = cx * inv_c, cy * inv_c, cz * inv_c
    vx = dy * cz - dz * cy
    vy = dz * cx - dx * cz
    vz = dx * cy - dy * cx

    ci0 = dx * rx + dy * ry + dz * rz
    ci1 = jnp.abs(cx * rx + cy * ry + cz * rz)
    ci2 = vx * rx + vy * ry + vz * rz
    cj0 = dx * qx + dy * qy + dz * qz
    cj1 = jnp.abs(cx * qx + cy * qy + cz * qz)
    cj2 = vx * qx + vy * qy + vz * qz

    dotp = ci0 * cj0 + ci1 * cj1 + ci2 * cj2
    ni = jnp.sqrt(ci0 * ci0 + ci1 * ci1 + ci2 * ci2)
    nj = jnp.sqrt(cj0 * cj0 + cj1 * cj1 + cj2 * cj2)
    pcos = dotp * pl.reciprocal((ni + EPSILON) * (nj + EPSILON), approx=True)
    pcos = jnp.clip(pcos, -1.0, 1.0)
    psin = jnp.sqrt(jnp.maximum(1.0 - pcos * pcos, 0.0))

    fw = fw_ref[...]

    def feats(c0, c2):
        s0, q0 = _sincos_2pi(c0 * fw)
        s2, q2 = _sincos_2pi(c2 * fw)
        return jnp.concatenate([s0, q0, s2, q2], axis=1).astype(bf16)

    p = (psin * wsin_ref[...] + pcos * wcos_ref[...]
         + jnp.dot(feats(ci0, ci2), wi_ref[...], preferred_element_type=f32)
         + jnp.dot(feats(cj0, cj2), wj_ref[...], preferred_element_type=f32)
         + pb1_ref[...])
    p = jax.nn.silu(p)
    p = jnp.dot(p.astype(bf16), pw2_ref[...],
                preferred_element_type=f32) + pb2_ref[...]

    ea_ref[...] = (edge2d + p).astype(ea_ref.dtype)
    te = dx.shape[0]
    basis_ref[...] = jnp.concatenate(
        [dx, dy, dz, cx, cy, cz, vx, vy, vz, jnp.zeros((te, 7), f32)],
        axis=1)


def _conv_kernel(h_ref, agg_ref, w1_ref, b1_ref, w2_ref, b2_ref, out_ref):
    z = h_ref[...] + agg_ref[...]
    z = jnp.dot(z.astype(bf16), w1_ref[...], preferred_element_type=f32) + b1_ref[...]
    z = jax.nn.silu(z)
    z = jnp.dot(z.astype(bf16), w2_ref[...], preferred_element_type=f32) + b2_ref[...]
    out_ref[...] = z


def _conv_last_kernel(h_ref, agg_ref, w1_ref, b1_ref, w2_ref, b2_ref,
                      dw_ref, c_ref):
    z = h_ref[...] + agg_ref[...]
    z = jnp.dot(z.astype(bf16), w1_ref[...], preferred_element_type=f32) + b1_ref[...]
    z = jax.nn.silu(z)
    z = jnp.dot(z.astype(bf16), w2_ref[...], preferred_element_type=f32) + b2_ref[...]
    c_ref[...] = jnp.dot(z.astype(bf16), dw_ref[...], preferred_element_type=f32)


def _mix_kernel(csum_ref, ea_ref, basis_ref,
                w1b_ref, b1_ref, w2_ref, b2_ref, mix_ref):
    f = (csum_ref[...]
         + jnp.dot(ea_ref[...], w1b_ref[...], preferred_element_type=f32)
         + b1_ref[...])
    f = jax.nn.silu(f)
    coff = jnp.dot(f.astype(bf16), w2_ref[...], preferred_element_type=f32) + b2_ref[...]
    b = basis_ref[...]
    mix_ref[...] = (coff[:, 0:1] * b[:, 0:3]
                    + coff[:, 1:2] * b[:, 3:6]
                    + coff[:, 2:3] * b[:, 6:9])


def kernel(node_2D_repr, positions, edge_index, node2graph, fwd_key,
           node_w, node_b, e2d_w1, e2d_b1, bn_g, bn_b, e2d_w2, e2d_b2,
           four_w, coff_w, coff_b, pj_wsin, pj_wcos, pj_wij, pj_b1,
           pj_w2, pj_b2, conv_w1, conv_b1, conv_w2, conv_b2,
           dyn_w1, dyn_b1, dyn_w2, dyn_b2):
    num_diffusion_timesteps = 1000
    sigma_min, sigma_max = 0.1, 1.0
    G = 256
    EDGE_TILE, NODE_TILE = 2048, 1024

    N, D = node_2D_repr.shape
    H = node_w.shape[1]
    HC = dyn_w1.shape[1]

    row = edge_index[0].astype(jnp.int32)
    col = edge_index[1].astype(jnp.int32)
    E = int(row.shape[0])

    row, col = jax.lax.sort((row, col), num_keys=1)

    key = jax.random.wrap_key_data(fwd_key)
    k_noise, k_time = jax.random.split(key)
    pos_noise = jax.random.normal(k_noise, positions.shape).astype(f32)
    half = G // 2 + 1
    ts = jax.random.randint(k_time, (half,), 0, num_diffusion_timesteps)
    ts = jnp.concatenate([ts, num_diffusion_timesteps - ts - 1])[:G]
    ts = ts.astype(f32) / num_diffusion_timesteps * (1.0 - EPSILON) + EPSILON
    t_pos = ts[node2graph]
    std_pos = sigma_min * (sigma_max / sigma_min) ** t_pos
    pos_perturbed = positions.astype(f32) + std_pos[:, None] * pos_noise

    TE = min(EDGE_TILE, _ceil_to(E, 16))
    E_pad = _ceil_to(E, TE)
    TN = min(NODE_TILE, _ceil_to(N + 1, 16))
    N_pad = _ceil_to(N + 1, TN)
    dummy = N_pad - 1
    n_et = E_pad // TE
    n_nt = N_pad // TN

    row_g = jnp.concatenate([row, jnp.full((E_pad - E,), dummy, jnp.int32)])
    col_g = jnp.concatenate([col, jnp.full((E_pad - E,), dummy, jnp.int32)])

    x_pad = jnp.zeros((N_pad, D), f32).at[:N].set(node_2D_repr.astype(f32))
    pos_pad = jnp.zeros((N_pad, 3), f32).at[:N].set(pos_perturbed)
    nedge = jnp.array([E], jnp.int32)

    a_tab, b_tab, h = pl.pallas_call(
        _node_pre_kernel,
        out_shape=(jax.ShapeDtypeStruct((N_pad, 2 * D), f32),
                   jax.ShapeDtypeStruct((N_pad, 2 * D), f32),
                   jax.ShapeDtypeStruct((N_pad, H), f32)),
        grid=(n_nt,),
        in_specs=[_rows(TN, D), _rows(TN, 3),
                  _rep((D, D)), _rep((D, D)), _rep((1, D)),
                  _rep((D, H)), _rep((1, H))],
        out_specs=(_rows(TN, 2 * D), _rows(TN, 2 * D), _rows(TN, H)),
        compiler_params=_CP,
    )(x_pad, pos_pad, e2d_w1[:D], e2d_w1[D:], e2d_b1,
      node_w.astype(bf16), node_b)

    hp = a_tab[row_g] + b_tab[col_g]

    stats = pl.pallas_call(
        _bn_stats_kernel,
        out_shape=jax.ShapeDtypeStruct((n_et * 8, D), f32),
        grid=(n_et,),
        in_specs=[pl.BlockSpec(memory_space=pltpu.MemorySpace.SMEM),
                  _rows(TE, D)],
        out_specs=pl.BlockSpec((8, D), lambda i: (i, 0)),
        compiler_params=_CP,
    )(nedge, hp)
    st = stats.reshape(n_et, 8, D)
    e_f = float(E)
    mu = (jnp.sum(st[:, 0, :], axis=0) / e_f)[None, :]
    var = jnp.maximum(jnp.sum(st[:, 1, :], axis=0) / e_f - mu[0] ** 2, 0.0)[None, :]
    bn_scale = bn_g * jax.lax.rsqrt(var + BN_EPS)
    bn_shift = bn_b - mu * bn_scale

    wij = pj_wij
    pj_wi = (coff_w @ wij[:H]).astype(bf16)
    pj_wj = (coff_w @ wij[H:]).astype(bf16)
    pj_b1f = coff_b @ wij[:H] + coff_b @ wij[H:] + pj_b1

    edge_attr, basis = pl.pallas_call(
        _edge_kernel,
        out_shape=(jax.ShapeDtypeStruct((E_pad, H), bf16),
                   jax.ShapeDtypeStruct((E_pad, 16), f32)),
        grid=(n_et,),
        in_specs=[_rows(TE, 2 * D), _rep((1, D)), _rep((1, D)),
                  _rep((D, H)), _rep((1, H)), _rep((1, H)),
                  _rep((1, H)), _rep((1, H)),
                  _rep((4 * H, H)), _rep((4 * H, H)), _rep((1, H)),
                  _rep((H, H)), _rep((1, H))],
        out_specs=(_rows(TE, H), _rows(TE, 16)),
        compiler_params=_CP,
    )(hp, bn_scale, bn_shift,
      e2d_w2.astype(bf16), e2d_b2, four_w,
      pj_wsin, pj_wcos, pj_wi, pj_wj, pj_b1f,
      pj_w2.astype(bf16), pj_b2)

    num_convs = conv_w1.shape[0]
    c_tab = None
    for layer in range(num_convs):
        msg = jax.nn.silu(h[col_g] + edge_attr.astype(f32))
        agg = jax.ops.segment_sum(msg, row_g, num_segments=N_pad,
                                  indices_are_sorted=True)
        if layer < num_convs - 1:
            h = pl.pallas_call(
                _conv_kernel,
                out_shape=jax.ShapeDtypeStruct((N_pad, H), f32),
                grid=(n_nt,),
                in_specs=[_rows(TN, H), _rows(TN, H),
                          _rep((H, 2 * H)), _rep((1, 2 * H)),
                          _rep((2 * H, H)), _rep((1, H))],
                out_specs=_rows(TN, H),
                compiler_params=_CP,
            )(h, agg,
              conv_w1[layer].astype(bf16), conv_b1[layer],
              conv_w2[layer].astype(bf16), conv_b2[layer])
        else:
            c_tab = pl.pallas_call(
                _conv_last_kernel,
                out_shape=jax.ShapeDtypeStruct((N_pad, HC), f32),
                grid=(n_nt,),
                in_specs=[_rows(TN, H), _rows(TN, H),
                          _rep((H, 2 * H)), _rep((1, 2 * H)),
                          _rep((2 * H, H)), _rep((1, H)),
                          _rep((H, HC))],
                out_specs=_rows(TN, HC),
                compiler_params=_CP,
            )(h, agg,
              conv_w1[layer].astype(bf16), conv_b1[layer],
              conv_w2[layer].astype(bf16), conv_b2[layer],
              dyn_w1[:H].astype(bf16))

    csum = c_tab[row_g] + c_tab[col_g]
    mix = pl.pallas_call(
        _mix_kernel,
        out_shape=jax.ShapeDtypeStruct((E_pad, 3), f32),
        grid=(n_et,),
        in_specs=[_rows(TE, HC), _rows(TE, H), _rows(TE, 16),
                  _rep((H, HC)), _rep((1, HC)),
                  _rep((HC, 3)), _rep((1, 3))],
        out_specs=_rows(TE, 3),
        compiler_params=_CP,
    )(csum, edge_attr, basis,
      dyn_w1[H:].astype(bf16), dyn_b1,
      dyn_w2.astype(bf16), dyn_b2)

    scores = jax.ops.segment_sum(mix[:E], row, num_segments=N,
                                 indices_are_sorted=True)
    d = scores - pos_noise
    sq = jnp.sum(d * d, axis=-1)
    loss_node = sq * (std_pos ** 2.0)
    counts = jax.ops.segment_sum(jnp.ones((N,), f32), node2graph, num_segments=G)
    loss_graph = jax.ops.segment_sum(loss_node, node2graph, num_segments=G) / counts
    loss = jnp.mean(loss_graph)
    return loss, scores


# speedup vs baseline: 1.0801x; 1.0801x over previous
import numpy as np
import jax
import jax.numpy as jnp
from jax.experimental import pallas as pl
from jax.experimental.pallas import tpu as pltpu

EPSILON = 1e-6
BN_EPS = 1e-5

f32 = jnp.float32
bf16 = jnp.bfloat16

_CP = pltpu.CompilerParams(
    dimension_semantics=("parallel",),
    vmem_limit_bytes=64 * 1024 * 1024,
)

_SIN_C = (6.2830885050e+00, -4.1333250451e+01, 8.1400142117e+01,
          -7.4676222887e+01, 3.3168810291e+01)
_COS_C = (9.9999944371e-01, -1.9739034398e+01, 6.4930614506e+01,
          -8.5295987236e+01, 5.8912646156e+01, -2.1283194093e+01)


def _sincos_2pi(t):
    r = t - jnp.round(t)
    u = r * r
    s = jnp.float32(_SIN_C[-1])
    for c in _SIN_C[-2::-1]:
        s = s * u + jnp.float32(c)
    s = s * r
    c_ = jnp.float32(_COS_C[-1])
    for c in _COS_C[-2::-1]:
        c_ = c_ * u + jnp.float32(c)
    return s, c_


def _ceil_to(x, m):
    return ((x + m - 1) // m) * m


def _rows(tile, cols):
    return pl.BlockSpec((tile, cols), lambda i: (i, 0))


def _rep(shape):
    return pl.BlockSpec(shape, lambda i: (0, 0))


def _cross(a, b):
    ax, ay, az = a[:, 0:1], a[:, 1:2], a[:, 2:3]
    bx, by, bz = b[:, 0:1], b[:, 1:2], b[:, 2:3]
    return jnp.concatenate(
        [ay * bz - az * by, az * bx - ax * bz, ax * by - ay * bx], axis=1)


def _node_pre_kernel(x_ref, pos_ref, w1a_ref, w1b_ref, b1_ref, nw_ref, nb_ref,
                     a_ref, b_ref, h_ref):
    x = x_ref[...]
    pos = pos_ref[...]
    tn, d = x.shape
    a = jnp.dot(x, w1a_ref[...], preferred_element_type=f32) + b1_ref[...]
    b = jnp.dot(x, w1b_ref[...], preferred_element_type=f32)
    zr = jnp.zeros((tn, 125), f32)
    zc = jnp.zeros((tn, 4), f32)
    a_ref[...] = jnp.concatenate([a, pos, zr], axis=1)
    b_ref[...] = jnp.concatenate([b, zc, pos, zr[:, :121]], axis=1)
    h_ref[...] = jnp.dot(x.astype(bf16), nw_ref[...],
                         preferred_element_type=f32) + nb_ref[...]


def _bn_stats_kernel(nedge_ref, h1_ref, stats_ref):
    h1 = h1_ref[...]
    te, d = h1.shape
    base = pl.program_id(0) * te
    idx = base + jax.lax.broadcasted_iota(jnp.int32, (te, 1), 0)
    hm = jnp.where(idx < nedge_ref[0], h1, 0.0)
    s = jnp.sum(hm, axis=0, keepdims=True)
    ss = jnp.sum(hm * h1, axis=0, keepdims=True)
    stats_ref[...] = jnp.concatenate([s, ss, jnp.zeros((6, d), f32)], axis=0)


def _edge_kernel(hp_ref, scale_ref, shift_ref,
                 w2_ref, b2_ref, fw_ref,
                 wsin_ref, wcos_ref, wi_ref, wj_ref, pb1_ref,
                 pw2_ref, pb2_ref,
                 ea_ref, basis_ref):
    hp = hp_ref[...]
    d = scale_ref.shape[1]
    hb = jnp.maximum(hp[:, :d] * scale_ref[...] + shift_ref[...], 0.0)
    edge2d = jnp.dot(hb.astype(bf16), w2_ref[...],
                     preferred_element_type=f32) + b2_ref[...]

    pr = hp[:, d:d + 3]
    pc = hp[:, d + 4:d + 7]
    diff = pr - pc
    radial = jnp.sum(diff * diff, axis=1, keepdims=True)
    cross = _cross(pr, pc)
    diff = diff * pl.reciprocal(jnp.sqrt(radial) + EPSILON, approx=True)
    cross = cross * pl.reciprocal(
        jnp.sqrt(jnp.sum(cross * cross, axis=1, keepdims=True)) + EPSILON,
        approx=True)
    vert = _cross(diff, cross)

    ci0 = jnp.sum(diff * pr, axis=1, keepdims=True)
    ci1 = jnp.abs(jnp.sum(cross * pr, axis=1, keepdims=True))
    ci2 = jnp.sum(vert * pr, axis=1, keepdims=True)
    cj0 = jnp.sum(diff * pc, axis=1, keepdims=True)
    cj1 = jnp.abs(jnp.sum(cross * pc, axis=1, keepdims=True))
    cj2 = jnp.sum(vert * pc, axis=1, keepdims=True)

    dotp = ci0 * cj0 + ci1 * cj1 + ci2 * cj2
    ni = jnp.sqrt(ci0 * ci0 + ci1 * ci1 + ci2 * ci2)
    nj = jnp.sqrt(cj0 * cj0 + cj1 * cj1 + cj2 * cj2)
    pcos = dotp * pl.reciprocal((ni + EPSILON) * (nj + EPSILON), approx=True)
    pcos = jnp.clip(pcos, -1.0, 1.0)
    psin = jnp.sqrt(jnp.maximum(1.0 - pcos * pcos, 0.0))

    fw = fw_ref[...]

    def feats(c0, c2):
        s0, q0 = _sincos_2pi(c0 * fw)
        s2, q2 = _sincos_2pi(c2 * fw)
        return jnp.concatenate([s0, q0, s2, q2], axis=1).astype(bf16)

    p = (psin * wsin_ref[...] + pcos * wcos_ref[...]
         + jnp.dot(feats(ci0, ci2), wi_ref[...], preferred_element_type=f32)
         + jnp.dot(feats(cj0, cj2), wj_ref[...], preferred_element_type=f32)
         + pb1_ref[...])
    p = jax.nn.silu(p)
    p = jnp.dot(p.astype(bf16), pw2_ref[...],
                preferred_element_type=f32) + pb2_ref[...]

    ea_ref[...] = (edge2d + p).astype(ea_ref.dtype)
    te = diff.shape[0]
    basis_ref[...] = jnp.concatenate(
        [diff, cross, vert, jnp.zeros((te, 7), f32)], axis=1)


def _conv_kernel(h_ref, agg_ref, w1_ref, b1_ref, w2_ref, b2_ref, out_ref):
    z = h_ref[...] + agg_ref[...]
    z = jnp.dot(z.astype(bf16), w1_ref[...], preferred_element_type=f32) + b1_ref[...]
    z = jax.nn.silu(z)
    z = jnp.dot(z.astype(bf16), w2_ref[...], preferred_element_type=f32) + b2_ref[...]
    out_ref[...] = z


def _conv_last_kernel(h_ref, agg_ref, w1_ref, b1_ref, w2_ref, b2_ref,
                      dw_ref, c_ref):
    z = h_ref[...] + agg_ref[...]
    z = jnp.dot(z.astype(bf16), w1_ref[...], preferred_element_type=f32) + b1_ref[...]
    z = jax.nn.silu(z)
    z = jnp.dot(z.astype(bf16), w2_ref[...], preferred_element_type=f32) + b2_ref[...]
    c_ref[...] = jnp.dot(z.astype(bf16), dw_ref[...], preferred_element_type=f32)


def _mix_kernel(csum_ref, ea_ref, basis_ref,
                w1b_ref, b1_ref, w2_ref, b2_ref, mix_ref):
    f = (csum_ref[...]
         + jnp.dot(ea_ref[...], w1b_ref[...], preferred_element_type=f32)
         + b1_ref[...])
    f = jax.nn.silu(f)
    coff = jnp.dot(f.astype(bf16), w2_ref[...], preferred_element_type=f32) + b2_ref[...]
    b = basis_ref[...]
    mix_ref[...] = (coff[:, 0:1] * b[:, 0:3]
                    + coff[:, 1:2] * b[:, 3:6]
                    + coff[:, 2:3] * b[:, 6:9])


def kernel(node_2D_repr, positions, edge_index, node2graph, fwd_key,
           node_w, node_b, e2d_w1, e2d_b1, bn_g, bn_b, e2d_w2, e2d_b2,
           four_w, coff_w, coff_b, pj_wsin, pj_wcos, pj_wij, pj_b1,
           pj_w2, pj_b2, conv_w1, conv_b1, conv_w2, conv_b2,
           dyn_w1, dyn_b1, dyn_w2, dyn_b2):
    num_diffusion_timesteps = 1000
    sigma_min, sigma_max = 0.1, 1.0
    G = 256
    EDGE_TILE, NODE_TILE = 4096, 1024

    N, D = node_2D_repr.shape
    H = node_w.shape[1]
    HC = dyn_w1.shape[1]

    row = edge_index[0].astype(jnp.int32)
    col = edge_index[1].astype(jnp.int32)
    E = int(row.shape[0])

    row, col = jax.lax.sort((row, col), num_keys=1)

    key = jax.random.wrap_key_data(fwd_key)
    k_noise, k_time = jax.random.split(key)
    pos_noise = jax.random.normal(k_noise, positions.shape).astype(f32)
    half = G // 2 + 1
    ts = jax.random.randint(k_time, (half,), 0, num_diffusion_timesteps)
    ts = jnp.concatenate([ts, num_diffusion_timesteps - ts - 1])[:G]
    ts = ts.astype(f32) / num_diffusion_timesteps * (1.0 - EPSILON) + EPSILON
    t_pos = ts[node2graph]
    std_pos = sigma_min * (sigma_max / sigma_min) ** t_pos
    pos_perturbed = positions.astype(f32) + std_pos[:, None] * pos_noise

    TE = min(EDGE_TILE, _ceil_to(E, 16))
    E_pad = _ceil_to(E, TE)
    TN = min(NODE_TILE, _ceil_to(N + 1, 16))
    N_pad = _ceil_to(N + 1, TN)
    dummy = N_pad - 1
    n_et = E_pad // TE
    n_nt = N_pad // TN

    row_g = jnp.concatenate([row, jnp.full((E_pad - E,), dummy, jnp.int32)])
    col_g = jnp.concatenate([col, jnp.full((E_pad - E,), dummy, jnp.int32)])

    x_pad = jnp.zeros((N_pad, D), f32).at[:N].set(node_2D_repr.astype(f32))
    pos_pad = jnp.zeros((N_pad, 3), f32).at[:N].set(pos_perturbed)
    nedge = jnp.array([E], jnp.int32)

    a_tab, b_tab, h = pl.pallas_call(
        _node_pre_kernel,
        out_shape=(jax.ShapeDtypeStruct((N_pad, 2 * D), f32),
                   jax.ShapeDtypeStruct((N_pad, 2 * D), f32),
                   jax.ShapeDtypeStruct((N_pad, H), f32)),
        grid=(n_nt,),
        in_specs=[_rows(TN, D), _rows(TN, 3),
                  _rep((D, D)), _rep((D, D)), _rep((1, D)),
                  _rep((D, H)), _rep((1, H))],
        out_specs=(_rows(TN, 2 * D), _rows(TN, 2 * D), _rows(TN, H)),
        compiler_params=_CP,
    )(x_pad, pos_pad, e2d_w1[:D], e2d_w1[D:], e2d_b1,
      node_w.astype(bf16), node_b)

    hp = a_tab[row_g] + b_tab[col_g]

    stats = pl.pallas_call(
        _bn_stats_kernel,
        out_shape=jax.ShapeDtypeStruct((n_et * 8, D), f32),
        grid=(n_et,),
        in_specs=[pl.BlockSpec(memory_space=pltpu.MemorySpace.SMEM),
                  _rows(TE, D)],
        out_specs=pl.BlockSpec((8, D), lambda i: (i, 0)),
        compiler_params=_CP,
    )(nedge, hp)
    st = stats.reshape(n_et, 8, D)
    e_f = float(E)
    mu = (jnp.sum(st[:, 0, :], axis=0) / e_f)[None, :]
    var = jnp.maximum(jnp.sum(st[:, 1, :], axis=0) / e_f - mu[0] ** 2, 0.0)[None, :]
    bn_scale = bn_g * jax.lax.rsqrt(var + BN_EPS)
    bn_shift = bn_b - mu * bn_scale

    wij = pj_wij
    pj_wi = (coff_w @ wij[:H]).astype(bf16)
    pj_wj = (coff_w @ wij[H:]).astype(bf16)
    pj_b1f = coff_b @ wij[:H] + coff_b @ wij[H:] + pj_b1

    edge_attr, basis = pl.pallas_call(
        _edge_kernel,
        out_shape=(jax.ShapeDtypeStruct((E_pad, H), bf16),
                   jax.ShapeDtypeStruct((E_pad, 16), f32)),
        grid=(n_et,),
        in_specs=[_rows(TE, 2 * D), _rep((1, D)), _rep((1, D)),
                  _rep((D, H)), _rep((1, H)), _rep((1, H)),
                  _rep((1, H)), _rep((1, H)),
                  _rep((4 * H, H)), _rep((4 * H, H)), _rep((1, H)),
                  _rep((H, H)), _rep((1, H))],
        out_specs=(_rows(TE, H), _rows(TE, 16)),
        compiler_params=_CP,
    )(hp, bn_scale, bn_shift,
      e2d_w2.astype(bf16), e2d_b2, four_w,
      pj_wsin, pj_wcos, pj_wi, pj_wj, pj_b1f,
      pj_w2.astype(bf16), pj_b2)

    num_convs = conv_w1.shape[0]
    c_tab = None
    for layer in range(num_convs):
        msg = jax.nn.silu(h[col_g] + edge_attr.astype(f32))
        agg = jax.ops.segment_sum(msg, row_g, num_segments=N_pad,
                                  indices_are_sorted=True)
        if layer < num_convs - 1:
            h = pl.pallas_call(
                _conv_kernel,
                out_shape=jax.ShapeDtypeStruct((N_pad, H), f32),
                grid=(n_nt,),
                in_specs=[_rows(TN, H), _rows(TN, H),
                          _rep((H, 2 * H)), _rep((1, 2 * H)),
                          _rep((2 * H, H)), _rep((1, H))],
                out_specs=_rows(TN, H),
                compiler_params=_CP,
            )(h, agg,
              conv_w1[layer].astype(bf16), conv_b1[layer],
              conv_w2[layer].astype(bf16), conv_b2[layer])
        else:
            c_tab = pl.pallas_call(
                _conv_last_kernel,
                out_shape=jax.ShapeDtypeStruct((N_pad, HC), f32),
                grid=(n_nt,),
                in_specs=[_rows(TN, H), _rows(TN, H),
                          _rep((H, 2 * H)), _rep((1, 2 * H)),
                          _rep((2 * H, H)), _rep((1, H)),
                          _rep((H, HC))],
                out_specs=_rows(TN, HC),
                compiler_params=_CP,
            )(h, agg,
              conv_w1[layer].astype(bf16), conv_b1[layer],
              conv_w2[layer].astype(bf16), conv_b2[layer],
              dyn_w1[:H].astype(bf16))

    csum = c_tab[row_g] + c_tab[col_g]
    mix = pl.pallas_call(
        _mix_kernel,
        out_shape=jax.ShapeDtypeStruct((E_pad, 3), f32),
        grid=(n_et,),
        in_specs=[_rows(TE, HC), _rows(TE, H), _rows(TE, 16),
                  _rep((H, HC)), _rep((1, HC)),
                  _rep((HC, 3)), _rep((1, 3))],
        out_specs=_rows(TE, 3),
        compiler_params=_CP,
    )(csum, edge_attr, basis,
      dyn_w1[H:].astype(bf16), dyn_b1,
      dyn_w2.astype(bf16), dyn_b2)

    scores = jax.ops.segment_sum(mix[:E], row, num_segments=N,
                                 indices_are_sorted=True)
    d = scores - pos_noise
    sq = jnp.sum(d * d, axis=-1)
    loss_node = sq * (std_pos ** 2.0)
    counts = jax.ops.segment_sum(jnp.ones((N,), f32), node2graph, num_segments=G)
    loss_graph = jax.ops.segment_sum(loss_node, node2graph, num_segments=G) / counts
    loss = jnp.mean(loss_graph)
    return loss, scores


# speedup vs baseline: 1.0862x; 1.0057x over previous
import numpy as np
import jax
import jax.numpy as jnp
from jax.experimental import pallas as pl
from jax.experimental.pallas import tpu as pltpu

EPSILON = 1e-6
BN_EPS = 1e-5

f32 = jnp.float32
bf16 = jnp.bfloat16

_CP = pltpu.CompilerParams(
    dimension_semantics=("parallel",),
    vmem_limit_bytes=64 * 1024 * 1024,
)

_SIN_C = (6.2830885050e+00, -4.1333250451e+01, 8.1400142117e+01,
          -7.4676222887e+01, 3.3168810291e+01)
_COS_C = (9.9999944371e-01, -1.9739034398e+01, 6.4930614506e+01,
          -8.5295987236e+01, 5.8912646156e+01, -2.1283194093e+01)


def _sincos_2pi(t):
    r = t - jnp.round(t)
    u = r * r
    s = jnp.float32(_SIN_C[-1])
    for c in _SIN_C[-2::-1]:
        s = s * u + jnp.float32(c)
    s = s * r
    c_ = jnp.float32(_COS_C[-1])
    for c in _COS_C[-2::-1]:
        c_ = c_ * u + jnp.float32(c)
    return s, c_


def _ceil_to(x, m):
    return ((x + m - 1) // m) * m


def _rows(tile, cols):
    return pl.BlockSpec((tile, cols), lambda i: (i, 0))


def _rep(shape):
    return pl.BlockSpec(shape, lambda i: (0, 0))


def _cross(a, b):
    ax, ay, az = a[:, 0:1], a[:, 1:2], a[:, 2:3]
    bx, by, bz = b[:, 0:1], b[:, 1:2], b[:, 2:3]
    return jnp.concatenate(
        [ay * bz - az * by, az * bx - ax * bz, ax * by - ay * bx], axis=1)


def _node_pre_kernel(x_ref, pos_ref, w1a_ref, w1b_ref, b1_ref, nw_ref, nb_ref,
                     a_ref, b_ref, h_ref):
    x = x_ref[...]
    pos = pos_ref[...]
    tn, d = x.shape
    a = jnp.dot(x, w1a_ref[...], preferred_element_type=f32) + b1_ref[...]
    b = jnp.dot(x, w1b_ref[...], preferred_element_type=f32)
    zr = jnp.zeros((tn, 125), f32)
    zc = jnp.zeros((tn, 4), f32)
    a_ref[...] = jnp.concatenate([a, pos, zr], axis=1)
    b_ref[...] = jnp.concatenate([b, zc, pos, zr[:, :121]], axis=1)
    h_ref[...] = jnp.dot(x.astype(bf16), nw_ref[...],
                         preferred_element_type=f32) + nb_ref[...]


def _bn_stats_kernel(nedge_ref, h1_ref, stats_ref):
    h1 = h1_ref[...]
    te, d = h1.shape
    base = pl.program_id(0) * te
    idx = base + jax.lax.broadcasted_iota(jnp.int32, (te, 1), 0)
    hm = jnp.where(idx < nedge_ref[0], h1, 0.0)
    s = jnp.sum(hm, axis=0, keepdims=True)
    ss = jnp.sum(hm * h1, axis=0, keepdims=True)
    stats_ref[...] = jnp.concatenate([s, ss, jnp.zeros((6, d), f32)], axis=0)


def _edge_kernel(hp_ref, scale_ref, shift_ref,
                 w2_ref, b2_ref, fw_ref,
                 wsin_ref, wcos_ref, wi_ref, wj_ref, pb1_ref,
                 pw2_ref, pb2_ref,
                 ea_ref, basis_ref):
    hp = hp_ref[...]
    d = scale_ref.shape[1]
    hb = jnp.maximum(hp[:, :d] * scale_ref[...] + shift_ref[...], 0.0)
    edge2d = jnp.dot(hb.astype(bf16), w2_ref[...],
                     preferred_element_type=f32) + b2_ref[...]

    pr = hp[:, d:d + 3]
    pc = hp[:, d + 4:d + 7]
    diff = pr - pc
    radial = jnp.sum(diff * diff, axis=1, keepdims=True)
    cross = _cross(pr, pc)
    diff = diff * pl.reciprocal(jnp.sqrt(radial) + EPSILON, approx=True)
    cross = cross * pl.reciprocal(
        jnp.sqrt(jnp.sum(cross * cross, axis=1, keepdims=True)) + EPSILON,
        approx=True)
    vert = _cross(diff, cross)

    ci0 = jnp.sum(diff * pr, axis=1, keepdims=True)
    ci1 = jnp.abs(jnp.sum(cross * pr, axis=1, keepdims=True))
    ci2 = jnp.sum(vert * pr, axis=1, keepdims=True)
    cj0 = jnp.sum(diff * pc, axis=1, keepdims=True)
    cj1 = jnp.abs(jnp.sum(cross * pc, axis=1, keepdims=True))
    cj2 = jnp.sum(vert * pc, axis=1, keepdims=True)

    dotp = ci0 * cj0 + ci1 * cj1 + ci2 * cj2
    ni = jnp.sqrt(ci0 * ci0 + ci1 * ci1 + ci2 * ci2)
    nj = jnp.sqrt(cj0 * cj0 + cj1 * cj1 + cj2 * cj2)
    pcos = dotp * pl.reciprocal((ni + EPSILON) * (nj + EPSILON), approx=True)
    pcos = jnp.clip(pcos, -1.0, 1.0)
    psin = jnp.sqrt(jnp.maximum(1.0 - pcos * pcos, 0.0))

    fw = fw_ref[...]

    def feats(c0, c2):
        s0, q0 = _sincos_2pi(c0 * fw)
        s2, q2 = _sincos_2pi(c2 * fw)
        return jnp.concatenate([s0, q0, s2, q2], axis=1).astype(bf16)

    p = (psin * wsin_ref[...] + pcos * wcos_ref[...]
         + jnp.dot(feats(ci0, ci2), wi_ref[...], preferred_element_type=f32)
         + jnp.dot(feats(cj0, cj2), wj_ref[...], preferred_element_type=f32)
         + pb1_ref[...])
    p = jax.nn.silu(p)
    p = jnp.dot(p.astype(bf16), pw2_ref[...],
                preferred_element_type=f32) + pb2_ref[...]

    ea_ref[...] = (edge2d + p).astype(ea_ref.dtype)
    te = diff.shape[0]
    basis_ref[...] = jnp.concatenate(
        [diff, cross, vert, jnp.zeros((te, 7), f32)], axis=1)


def _conv_kernel(h_ref, agg_ref, w1_ref, b1_ref, w2_ref, b2_ref, out_ref):
    z = h_ref[...] + agg_ref[...]
    z = jnp.dot(z.astype(bf16), w1_ref[...], preferred_element_type=f32) + b1_ref[...]
    z = jax.nn.silu(z)
    z = jnp.dot(z.astype(bf16), w2_ref[...], preferred_element_type=f32) + b2_ref[...]
    out_ref[...] = z


def _conv_last_kernel(h_ref, agg_ref, w1_ref, b1_ref, w2_ref, b2_ref,
                      dw_ref, c_ref):
    z = h_ref[...] + agg_ref[...]
    z = jnp.dot(z.astype(bf16), w1_ref[...], preferred_element_type=f32) + b1_ref[...]
    z = jax.nn.silu(z)
    z = jnp.dot(z.astype(bf16), w2_ref[...], preferred_element_type=f32) + b2_ref[...]
    c_ref[...] = jnp.dot(z.astype(bf16), dw_ref[...], preferred_element_type=f32)


def _mix_kernel(csum_ref, ea_ref, basis_ref,
                w1b_ref, b1_ref, w2_ref, b2_ref, mix_ref):
    f = (csum_ref[...]
         + jnp.dot(ea_ref[...], w1b_ref[...], preferred_element_type=f32)
         + b1_ref[...])
    f = jax.nn.silu(f)
    coff = jnp.dot(f.astype(bf16), w2_ref[...], preferred_element_type=f32) + b2_ref[...]
    b = basis_ref[...]
    mix_ref[...] = (coff[:, 0:1] * b[:, 0:3]
                    + coff[:, 1:2] * b[:, 3:6]
                    + coff[:, 2:3] * b[:, 6:9])


def kernel(node_2D_repr, positions, edge_index, node2graph, fwd_key,
           node_w, node_b, e2d_w1, e2d_b1, bn_g, bn_b, e2d_w2, e2d_b2,
           four_w, coff_w, coff_b, pj_wsin, pj_wcos, pj_wij, pj_b1,
           pj_w2, pj_b2, conv_w1, conv_b1, conv_w2, conv_b2,
           dyn_w1, dyn_b1, dyn_w2, dyn_b2):
    num_diffusion_timesteps = 1000
    sigma_min, sigma_max = 0.1, 1.0
    G = 256
    EDGE_TILE, NODE_TILE = 4096, 2048

    N, D = node_2D_repr.shape
    H = node_w.shape[1]
    HC = dyn_w1.shape[1]

    row = edge_index[0].astype(jnp.int32)
    col = edge_index[1].astype(jnp.int32)
    E = int(row.shape[0])

    row, col = jax.lax.sort((row, col), num_keys=1)

    key = jax.random.wrap_key_data(fwd_key)
    k_noise, k_time = jax.random.split(key)
    pos_noise = jax.random.normal(k_noise, positions.shape).astype(f32)
    half = G // 2 + 1
    ts = jax.random.randint(k_time, (half,), 0, num_diffusion_timesteps)
    ts = jnp.concatenate([ts, num_diffusion_timesteps - ts - 1])[:G]
    ts = ts.astype(f32) / num_diffusion_timesteps * (1.0 - EPSILON) + EPSILON
    t_pos = ts[node2graph]
    std_pos = sigma_min * (sigma_max / sigma_min) ** t_pos
    pos_perturbed = positions.astype(f32) + std_pos[:, None] * pos_noise

    TE = min(EDGE_TILE, _ceil_to(E, 16))
    E_pad = _ceil_to(E, TE)
    TN = min(NODE_TILE, _ceil_to(N + 1, 16))
    N_pad = _ceil_to(N + 1, TN)
    dummy = N_pad - 1
    n_et = E_pad // TE
    n_nt = N_pad // TN

    row_g = jnp.concatenate([row, jnp.full((E_pad - E,), dummy, jnp.int32)])
    col_g = jnp.concatenate([col, jnp.full((E_pad - E,), dummy, jnp.int32)])

    x_pad = jnp.zeros((N_pad, D), f32).at[:N].set(node_2D_repr.astype(f32))
    pos_pad = jnp.zeros((N_pad, 3), f32).at[:N].set(pos_perturbed)
    nedge = jnp.array([E], jnp.int32)

    a_tab, b_tab, h = pl.pallas_call(
        _node_pre_kernel,
        out_shape=(jax.ShapeDtypeStruct((N_pad, 2 * D), f32),
                   jax.ShapeDtypeStruct((N_pad, 2 * D), f32),
                   jax.ShapeDtypeStruct((N_pad, H), f32)),
        grid=(n_nt,),
        in_specs=[_rows(TN, D), _rows(TN, 3),
                  _rep((D, D)), _rep((D, D)), _rep((1, D)),
                  _rep((D, H)), _rep((1, H))],
        out_specs=(_rows(TN, 2 * D), _rows(TN, 2 * D), _rows(TN, H)),
        compiler_params=_CP,
    )(x_pad, pos_pad, e2d_w1[:D], e2d_w1[D:], e2d_b1,
      node_w.astype(bf16), node_b)

    hp = a_tab[row_g] + b_tab[col_g]

    stats = pl.pallas_call(
        _bn_stats_kernel,
        out_shape=jax.ShapeDtypeStruct((n_et * 8, D), f32),
        grid=(n_et,),
        in_specs=[pl.BlockSpec(memory_space=pltpu.MemorySpace.SMEM),
                  _rows(TE, D)],
        out_specs=pl.BlockSpec((8, D), lambda i: (i, 0)),
        compiler_params=_CP,
    )(nedge, hp)
    st = stats.reshape(n_et, 8, D)
    e_f = float(E)
    mu = (jnp.sum(st[:, 0, :], axis=0) / e_f)[None, :]
    var = jnp.maximum(jnp.sum(st[:, 1, :], axis=0) / e_f - mu[0] ** 2, 0.0)[None, :]
    bn_scale = bn_g * jax.lax.rsqrt(var + BN_EPS)
    bn_shift = bn_b - mu * bn_scale

    wij = pj_wij
    pj_wi = (coff_w @ wij[:H]).astype(bf16)
    pj_wj = (coff_w @ wij[H:]).astype(bf16)
    pj_b1f = coff_b @ wij[:H] + coff_b @ wij[H:] + pj_b1

    edge_attr, basis = pl.pallas_call(
        _edge_kernel,
        out_shape=(jax.ShapeDtypeStruct((E_pad, H), bf16),
                   jax.ShapeDtypeStruct((E_pad, 16), f32)),
        grid=(n_et,),
        in_specs=[_rows(TE, 2 * D), _rep((1, D)), _rep((1, D)),
                  _rep((D, H)), _rep((1, H)), _rep((1, H)),
                  _rep((1, H)), _rep((1, H)),
                  _rep((4 * H, H)), _rep((4 * H, H)), _rep((1, H)),
                  _rep((H, H)), _rep((1, H))],
        out_specs=(_rows(TE, H), _rows(TE, 16)),
        compiler_params=_CP,
    )(hp, bn_scale, bn_shift,
      e2d_w2.astype(bf16), e2d_b2, four_w,
      pj_wsin, pj_wcos, pj_wi, pj_wj, pj_b1f,
      pj_w2.astype(bf16), pj_b2)

    num_convs = conv_w1.shape[0]
    c_tab = None
    for layer in range(num_convs):
        msg = jax.nn.silu(h[col_g] + edge_attr.astype(f32))
        agg = jax.ops.segment_sum(msg, row_g, num_segments=N_pad,
                                  indices_are_sorted=True)
        if layer < num_convs - 1:
            h = pl.pallas_call(
                _conv_kernel,
                out_shape=jax.ShapeDtypeStruct((N_pad, H), f32),
                grid=(n_nt,),
                in_specs=[_rows(TN, H), _rows(TN, H),
                          _rep((H, 2 * H)), _rep((1, 2 * H)),
                          _rep((2 * H, H)), _rep((1, H))],
                out_specs=_rows(TN, H),
                compiler_params=_CP,
            )(h, agg,
              conv_w1[layer].astype(bf16), conv_b1[layer],
              conv_w2[layer].astype(bf16), conv_b2[layer])
        else:
            c_tab = pl.pallas_call(
                _conv_last_kernel,
                out_shape=jax.ShapeDtypeStruct((N_pad, HC), f32),
                grid=(n_nt,),
                in_specs=[_rows(TN, H), _rows(TN, H),
                          _rep((H, 2 * H)), _rep((1, 2 * H)),
                          _rep((2 * H, H)), _rep((1, H)),
                          _rep((H, HC))],
                out_specs=_rows(TN, HC),
                compiler_params=_CP,
            )(h, agg,
              conv_w1[layer].astype(bf16), conv_b1[layer],
              conv_w2[layer].astype(bf16), conv_b2[layer],
              dyn_w1[:H].astype(bf16))

    csum = c_tab[row_g] + c_tab[col_g]
    mix = pl.pallas_call(
        _mix_kernel,
        out_shape=jax.ShapeDtypeStruct((E_pad, 3), f32),
        grid=(n_et,),
        in_specs=[_rows(TE, HC), _rows(TE, H), _rows(TE, 16),
                  _rep((H, HC)), _rep((1, HC)),
                  _rep((HC, 3)), _rep((1, 3))],
        out_specs=_rows(TE, 3),
        compiler_params=_CP,
    )(csum, edge_attr, basis,
      dyn_w1[H:].astype(bf16), dyn_b1,
      dyn_w2.astype(bf16), dyn_b2)

    scores = jax.ops.segment_sum(mix[:E], row, num_segments=N,
                                 indices_are_sorted=True)
    d = scores - pos_noise
    sq = jnp.sum(d * d, axis=-1)
    loss_node = sq * (std_pos ** 2.0)
    counts = jax.ops.segment_sum(jnp.ones((N,), f32), node2graph, num_segments=G)
    loss_graph = jax.ops.segment_sum(loss_node, node2graph, num_segments=G) / counts
    loss = jnp.mean(loss_graph)
    return loss, scores
